# baseline (speedup 1.0000x reference)
import jax
import jax.numpy as jnp
from jax.experimental import pallas as pl
from jax.experimental.pallas import tpu as pltpu

IN_F = 128
HID_F = 128
OUT_F = 128
N_NGBRS = 320000

CHMAX = 16000
NBUF = 7
SIZES = [2048, 2048, 4096, 8192] + [16000] * 18 + [15616]
OFFS = []
_o = 0
for _s in SIZES:
    OFFS.append(_o)
    _o += _s
assert _o == N_NGBRS
NCHUNK = len(SIZES)

_SLOPE = 0.01


def _lrelu(x):
    return jnp.maximum(x, _SLOPE * x)


def _dot(a, b):
    return jax.lax.dot_general(
        a, b, (((1,), (0,)), ((), ())), preferred_element_type=jnp.float32)


def _pinsage_kernel(x_hbm, a_hbm, qt_ref, qb_ref, hn_ref, wt_ref, wb_ref,
                    out_ref, xbuf, abuf, xsem, asem):
    def _x_copy(c):
        b = c % NBUF
        return pltpu.make_async_copy(
            x_hbm.at[pl.ds(OFFS[c], SIZES[c]), :],
            xbuf.at[b, pl.ds(0, SIZES[c]), :],
            xsem.at[b])

    a_all = pltpu.make_async_copy(a_hbm, abuf, asem)
    a_all.start()
    for c in range(NBUF):
        _x_copy(c).start()
    a_all.wait()
    asum = jnp.sum(abuf[...])

    qt = qt_ref[...].astype(jnp.bfloat16)
    qb16 = qb_ref[...].astype(jnp.bfloat16)
    acc = jnp.zeros((1, HID_F), jnp.float32)
    for c in range(NCHUNK):
        b = c % NBUF
        _x_copy(c).wait()
        x = xbuf[b, 0:SIZES[c], :]
        a = abuf[:, OFFS[c]:OFFS[c] + SIZES[c]]
        z16 = _dot(x.astype(jnp.bfloat16), qt).astype(jnp.bfloat16) + qb16
        l16 = _lrelu(z16)
        acc = acc + _dot(a.astype(jnp.bfloat16), l16)
        if c + NBUF < NCHUNK:
            _x_copy(c + NBUF).start()

    ssafe = jnp.where(asum == 0.0, 1.0, asum)
    h_agg = acc / ssafe
    wt = wt_ref[...]
    z2 = _dot(hn_ref[...], wt[:IN_F, :]) + _dot(h_agg, wt[IN_F:, :]) \
        + wb_ref[...]
    h_two = _lrelu(z2)
    nrm = jnp.sqrt(jnp.sum(h_two * h_two))
    nsafe = jnp.where(nrm == 0.0, 1.0, nrm)
    out_ref[...] = h_two / nsafe


@jax.jit
def kernel(h_node, h_ngbrs, alpha, Q_w, Q_b, W_w, W_b):
    out = pl.pallas_call(
        _pinsage_kernel,
        in_specs=[
            pl.BlockSpec(memory_space=pltpu.MemorySpace.HBM),
            pl.BlockSpec(memory_space=pltpu.MemorySpace.HBM),
            pl.BlockSpec(memory_space=pltpu.MemorySpace.VMEM),
            pl.BlockSpec(memory_space=pltpu.MemorySpace.VMEM),
            pl.BlockSpec(memory_space=pltpu.MemorySpace.VMEM),
            pl.BlockSpec(memory_space=pltpu.MemorySpace.VMEM),
            pl.BlockSpec(memory_space=pltpu.MemorySpace.VMEM),
        ],
        out_specs=pl.BlockSpec(memory_space=pltpu.MemorySpace.VMEM),
        out_shape=jax.ShapeDtypeStruct((1, OUT_F), jnp.float32),
        compiler_params=pltpu.CompilerParams(
            vmem_limit_bytes=128 * 1024 * 1024),
        scratch_shapes=[
            pltpu.VMEM((NBUF, CHMAX, IN_F), jnp.float32),
            pltpu.VMEM((1, N_NGBRS), jnp.float32),
            pltpu.SemaphoreType.DMA((NBUF,)),
            pltpu.SemaphoreType.DMA(()),
        ],
    )(
        h_ngbrs,
        alpha.reshape(1, N_NGBRS),
        Q_w.T,
        Q_b.reshape(1, HID_F),
        h_node.reshape(1, IN_F),
        W_w.T,
        W_b.reshape(1, OUT_F),
    )
    return out[0]


# speedup vs baseline: 1.0349x; 1.0349x over previous
import jax
import jax.numpy as jnp
from jax.experimental import pallas as pl
from jax.experimental.pallas import tpu as pltpu

IN_F = 128
HID_F = 128
OUT_F = 128
N_NGBRS = 320000

CHMAX = 16000
NBUF = 7
SIZES = [2048, 2048, 4096, 8192] + [16000] * 18 + [15616]
OFFS = []
_o = 0
for _s in SIZES:
    OFFS.append(_o)
    _o += _s
assert _o == N_NGBRS
NCHUNK = len(SIZES)

_SLOPE = 0.01


def _lrelu(x):
    return jnp.maximum(x, _SLOPE * x)


def _dot(a, b):
    return jax.lax.dot_general(
        a, b, (((1,), (0,)), ((), ())), preferred_element_type=jnp.float32)


def _pinsage_kernel(x_hbm, a_hbm, qt_ref, qb_ref, hn_ref, wt_ref, wb_ref,
                    out_ref, xbuf, abuf, xsem, asem):
    def _x_copy(c):
        b = c % NBUF
        return pltpu.make_async_copy(
            x_hbm.at[pl.ds(OFFS[c], SIZES[c]), :],
            xbuf.at[b, pl.ds(0, SIZES[c]), :],
            xsem.at[b])

    a_all = pltpu.make_async_copy(a_hbm, abuf, asem)
    a_all.start()
    for c in range(NBUF):
        _x_copy(c).start()
    a_all.wait()

    qt = qt_ref[...].astype(jnp.bfloat16)
    qb16 = qb_ref[...].astype(jnp.bfloat16)
    acc = jnp.zeros((1, HID_F), jnp.float32)
    for c in range(NCHUNK):
        b = c % NBUF
        _x_copy(c).wait()
        x = xbuf[b, 0:SIZES[c], :]
        a = abuf[:, OFFS[c]:OFFS[c] + SIZES[c]]
        z16 = _dot(x.astype(jnp.bfloat16), qt).astype(jnp.bfloat16) + qb16
        l16 = _lrelu(z16)
        acc = acc + _dot(a.astype(jnp.bfloat16), l16)
        if c + NBUF < NCHUNK:
            _x_copy(c + NBUF).start()

    asum = jnp.sum(abuf[...])
    ssafe = jnp.where(asum == 0.0, 1.0, asum)
    h_agg = acc / ssafe
    wt = wt_ref[...]
    z2 = _dot(hn_ref[...], wt[:IN_F, :]) + _dot(h_agg, wt[IN_F:, :]) \
        + wb_ref[...]
    h_two = _lrelu(z2)
    nrm = jnp.sqrt(jnp.sum(h_two * h_two))
    nsafe = jnp.where(nrm == 0.0, 1.0, nrm)
    out_ref[...] = h_two / nsafe


@jax.jit
def kernel(h_node, h_ngbrs, alpha, Q_w, Q_b, W_w, W_b):
    out = pl.pallas_call(
        _pinsage_kernel,
        in_specs=[
            pl.BlockSpec(memory_space=pltpu.MemorySpace.HBM),
            pl.BlockSpec(memory_space=pltpu.MemorySpace.HBM),
            pl.BlockSpec(memory_space=pltpu.MemorySpace.VMEM),
            pl.BlockSpec(memory_space=pltpu.MemorySpace.VMEM),
            pl.BlockSpec(memory_space=pltpu.MemorySpace.VMEM),
            pl.BlockSpec(memory_space=pltpu.MemorySpace.VMEM),
            pl.BlockSpec(memory_space=pltpu.MemorySpace.VMEM),
        ],
        out_specs=pl.BlockSpec(memory_space=pltpu.MemorySpace.VMEM),
        out_shape=jax.ShapeDtypeStruct((1, OUT_F), jnp.float32),
        compiler_params=pltpu.CompilerParams(
            vmem_limit_bytes=128 * 1024 * 1024),
        scratch_shapes=[
            pltpu.VMEM((NBUF, CHMAX, IN_F), jnp.float32),
            pltpu.VMEM((1, N_NGBRS), jnp.float32),
            pltpu.SemaphoreType.DMA((NBUF,)),
            pltpu.SemaphoreType.DMA(()),
        ],
    )(
        h_ngbrs,
        alpha.reshape(1, N_NGBRS),
        Q_w.T,
        Q_b.reshape(1, HID_F),
        h_node.reshape(1, IN_F),
        W_w.T,
        W_b.reshape(1, OUT_F),
    )
    return out[0]


# speedup vs baseline: 1.1114x; 1.0739x over previous
import jax
import jax.numpy as jnp
from jax.experimental import pallas as pl
from jax.experimental.pallas import tpu as pltpu

IN_F = 128
HID_F = 128
OUT_F = 128
N_NGBRS = 320000

CHMAX = 16000
NBUF = 7
SIZES = [2048, 2048, 4096, 8192] + [16000] * 18 + [15616]
OFFS = []
_o = 0
for _s in SIZES:
    OFFS.append(_o)
    _o += _s
assert _o == N_NGBRS
NCHUNK = len(SIZES)

_SLOPE = 0.01


def _lrelu(x):
    return jnp.maximum(x, _SLOPE * x)


def _dot(a, b):
    return jax.lax.dot_general(
        a, b, (((1,), (0,)), ((), ())), preferred_element_type=jnp.float32)


def _pinsage_kernel(x_hbm, a_hbm, qt_ref, qb_ref, hn_ref, wt_ref, wb_ref,
                    out_ref, xbuf, abuf, xsem, asem):
    def _x_copy(c):
        b = c % NBUF
        return pltpu.make_async_copy(
            x_hbm.at[pl.ds(OFFS[c], SIZES[c]), :],
            xbuf.at[b, pl.ds(0, SIZES[c]), :],
            xsem.at[b])

    def _a_copy(c):
        b = c % NBUF
        return pltpu.make_async_copy(
            a_hbm.at[:, pl.ds(OFFS[c], SIZES[c])],
            abuf.at[b, :, pl.ds(0, SIZES[c])],
            asem.at[b])

    for c in range(NBUF):
        _x_copy(c).start()
        _a_copy(c).start()

    qt = qt_ref[...].astype(jnp.bfloat16)
    qb16 = qb_ref[...].astype(jnp.bfloat16)
    acc = jnp.zeros((1, HID_F), jnp.float32)
    asum = jnp.float32(0.0)
    for c in range(NCHUNK):
        b = c % NBUF
        _x_copy(c).wait()
        _a_copy(c).wait()
        x = xbuf[b, 0:SIZES[c], :]
        a = abuf[b, :, 0:SIZES[c]]
        z16 = _dot(x.astype(jnp.bfloat16), qt).astype(jnp.bfloat16) + qb16
        l16 = _lrelu(z16)
        acc = acc + _dot(a.astype(jnp.bfloat16), l16)
        asum = asum + jnp.sum(a)
        if c + NBUF < NCHUNK:
            _x_copy(c + NBUF).start()
            _a_copy(c + NBUF).start()

    ssafe = jnp.where(asum == 0.0, 1.0, asum)
    h_agg = acc / ssafe
    wt = wt_ref[...]
    z2 = _dot(hn_ref[...], wt[:IN_F, :]) + _dot(h_agg, wt[IN_F:, :]) \
        + wb_ref[...]
    h_two = _lrelu(z2)
    nrm = jnp.sqrt(jnp.sum(h_two * h_two))
    nsafe = jnp.where(nrm == 0.0, 1.0, nrm)
    out_ref[...] = h_two / nsafe


@jax.jit
def kernel(h_node, h_ngbrs, alpha, Q_w, Q_b, W_w, W_b):
    out = pl.pallas_call(
        _pinsage_kernel,
        in_specs=[
            pl.BlockSpec(memory_space=pltpu.MemorySpace.HBM),
            pl.BlockSpec(memory_space=pltpu.MemorySpace.HBM),
            pl.BlockSpec(memory_space=pltpu.MemorySpace.VMEM),
            pl.BlockSpec(memory_space=pltpu.MemorySpace.VMEM),
            pl.BlockSpec(memory_space=pltpu.MemorySpace.VMEM),
            pl.BlockSpec(memory_space=pltpu.MemorySpace.VMEM),
            pl.BlockSpec(memory_space=pltpu.MemorySpace.VMEM),
        ],
        out_specs=pl.BlockSpec(memory_space=pltpu.MemorySpace.VMEM),
        out_shape=jax.ShapeDtypeStruct((1, OUT_F), jnp.float32),
        compiler_params=pltpu.CompilerParams(
            vmem_limit_bytes=128 * 1024 * 1024),
        scratch_shapes=[
            pltpu.VMEM((NBUF, CHMAX, IN_F), jnp.float32),
            pltpu.VMEM((NBUF, 1, CHMAX), jnp.float32),
            pltpu.SemaphoreType.DMA((NBUF,)),
            pltpu.SemaphoreType.DMA((NBUF,)),
        ],
    )(
        h_ngbrs,
        alpha.reshape(1, N_NGBRS),
        Q_w.T,
        Q_b.reshape(1, HID_F),
        h_node.reshape(1, IN_F),
        W_w.T,
        W_b.reshape(1, OUT_F),
    )
    return out[0]
